# baseline (speedup 1.0000x reference)
import math

import jax
import jax.numpy as jnp
from jax import lax
from jax.experimental import pallas as pl
from jax.experimental.pallas import tpu as pltpu
from jax.experimental.pallas import tpu_sc as plsc

_NUM_USERS = 100000
_POOL_SIZE = 200
_NUM_NEG = 20
_BATCH = 16384

_NC = 2
_NS = 16
_L = 16
_NW = _NC * _NS
_BPW = _BATCH // _NW
_EPW = _BPW * _NUM_NEG
_CHUNK = 128
_NCH = _EPW // _CHUNK
_VPC = _CHUNK // _L


def _tec_body(user_hbm, pool_hbm, idxk_hbm, out_hbm,
              user_v, idx_v, flat_v, out_v, sem):
    wid = lax.axis_index("s") * _NC + lax.axis_index("c")
    base = wid * _BPW
    ebase = base * _NUM_NEG

    pltpu.sync_copy(user_hbm.at[pl.ds(base, _BPW)], user_v)
    pltpu.sync_copy(idxk_hbm.at[pl.ds(ebase, _EPW)], idx_v)

    iota = lax.iota(jnp.int32, _L)

    def chunk_body(c, carry):
        def vec_body(e, carry2):
            o = c * _CHUNK + e * _L
            lanes = o + iota
            b_loc = lax.div(lanes, jnp.int32(_NUM_NEG))
            users = plsc.load_gather(user_v, [b_loc])
            col = idx_v[pl.ds(o, _L)]
            flat_v[pl.ds(o, _L)] = users * _POOL_SIZE + col
            return carry2

        lax.fori_loop(0, _VPC, vec_body, 0)
        pltpu.async_copy(
            pool_hbm.at[flat_v.at[pl.ds(c * _CHUNK, _CHUNK)]],
            out_v.at[pl.ds(c * _CHUNK, _CHUNK)],
            sem,
        )
        return carry

    lax.fori_loop(0, _NCH, chunk_body, 0)

    pltpu.make_async_copy(pool_hbm.at[pl.ds(0, _EPW)], out_v, sem).wait()

    pltpu.sync_copy(out_v, out_hbm.at[pl.ds(ebase, _EPW)])


def kernel(user_id, pool, idx_k):
    mesh = plsc.VectorSubcoreMesh(core_axis_name="c", subcore_axis_name="s")
    kfn = pl.kernel(
        _tec_body,
        mesh=mesh,
        compiler_params=pltpu.CompilerParams(
            use_tc_tiling_on_sc=False, needs_layout_passes=False),
        out_type=jax.ShapeDtypeStruct((_BATCH * _NUM_NEG,), jnp.int32),
        scratch_types=[
            pltpu.VMEM((_BPW,), jnp.int32),
            pltpu.VMEM((_EPW,), jnp.int32),
            pltpu.VMEM((_EPW,), jnp.int32),
            pltpu.VMEM((_EPW,), jnp.int32),
            pltpu.SemaphoreType.DMA,
        ],
    )
    neg_flat = kfn(user_id, pool.reshape(-1), idx_k.reshape(-1))
    neg_items = neg_flat.reshape(_BATCH, _NUM_NEG)
    log_q = jnp.full((_BATCH, _NUM_NEG), -math.log(float(_POOL_SIZE)),
                     dtype=jnp.float32)
    return neg_items, log_q


# speedup vs baseline: 1.0002x; 1.0002x over previous
import math

import jax
import jax.numpy as jnp
from jax import lax
from jax.experimental import pallas as pl
from jax.experimental.pallas import tpu as pltpu
from jax.experimental.pallas import tpu_sc as plsc

_NUM_USERS = 100000
_POOL_SIZE = 200
_NUM_NEG = 20
_BATCH = 16384

_NC = 2
_NS = 16
_L = 16
_NW = _NC * _NS
_BPW = _BATCH // _NW
_EPW = _BPW * _NUM_NEG
_CHUNK = 128
_NCH = _EPW // _CHUNK
_VPC = _CHUNK // _L


def _tec_body(user_hbm, pool_hbm, idxk_hbm, out_hbm,
              user_v, idx_v, flat_v, out_v, sem):
    wid = lax.axis_index("s") * _NC + lax.axis_index("c")
    base = wid * _BPW
    ebase = base * _NUM_NEG

    pltpu.sync_copy(user_hbm.at[pl.ds(base, _BPW)], user_v)
    pltpu.sync_copy(idxk_hbm.at[pl.ds(ebase, _EPW)], idx_v)

    iota = lax.iota(jnp.int32, _L)

    def chunk_body(c, carry):
        def vec_body(e, carry2):
            o = c * _CHUNK + e * _L
            lanes = o + iota
            b_loc = lax.div(lanes, jnp.int32(_NUM_NEG))
            users = plsc.load_gather(user_v, [b_loc])
            col = idx_v[pl.ds(o, _L)]
            flat_v[pl.ds(o, _L)] = users * _POOL_SIZE + col
            return carry2

        lax.fori_loop(0, _VPC, vec_body, 0)
        pltpu.async_copy(
            pool_hbm.at[flat_v.at[pl.ds(c * _CHUNK, _CHUNK)]],
            out_v.at[pl.ds(c * _CHUNK, _CHUNK)],
            sem,
        )
        return carry

    lax.fori_loop(0, _NCH, chunk_body, 0)

    pltpu.make_async_copy(pool_hbm.at[pl.ds(0, _EPW)], out_v, sem).wait()

    pltpu.sync_copy(out_v, out_hbm.at[pl.ds(ebase, _EPW)])


def kernel(user_id, pool, idx_k):
    mesh = plsc.VectorSubcoreMesh(core_axis_name="c", subcore_axis_name="s")
    kfn = pl.kernel(
        _tec_body,
        mesh=mesh,
        compiler_params=pltpu.CompilerParams(
            use_tc_tiling_on_sc=False, needs_layout_passes=False),
        out_type=jax.ShapeDtypeStruct((_BATCH * _NUM_NEG,), jnp.int32),
        scratch_types=[
            pltpu.VMEM((_BPW,), jnp.int32),
            pltpu.VMEM((_EPW,), jnp.int32),
            pltpu.VMEM((_EPW,), jnp.int32),
            pltpu.VMEM((_EPW,), jnp.int32),
            pltpu.SemaphoreType.DMA,
        ],
    )
    pool_lin = lax.optimization_barrier(pool.reshape(-1))
    neg_flat = kfn(user_id, pool_lin, idx_k.reshape(-1))
    neg_items = neg_flat.reshape(_BATCH, _NUM_NEG)
    log_q = jnp.full((_BATCH, _NUM_NEG), -math.log(float(_POOL_SIZE)),
                     dtype=jnp.float32)
    return neg_items, log_q


# speedup vs baseline: 3.9481x; 3.9472x over previous
import math

import jax
import jax.numpy as jnp
from jax import lax
from jax.experimental import pallas as pl
from jax.experimental.pallas import tpu as pltpu
from jax.experimental.pallas import tpu_sc as plsc

_NUM_USERS = 100000
_POOL_SIZE = 200
_NUM_NEG = 20
_BATCH = 16384

_NC = 2
_NS = 16
_L = 16
_NW = _NC * _NS
_TOT = _BATCH * _NUM_NEG
_EPW = _TOT // _NW
_CHUNK = 128
_NCH = _EPW // _CHUNK
_VPC = _CHUNK // _L


def _tec_body(user_hbm, pool_hbm, idxk_hbm, out_hbm,
              user_v, idx_v, flat_v, out_v, sem):
    wid = lax.axis_index("s") * _NC + lax.axis_index("c")
    ebase = wid * _EPW

    pltpu.sync_copy(user_hbm, user_v)
    pltpu.sync_copy(idxk_hbm.at[pl.ds(ebase, _EPW)], idx_v)

    iota = lax.iota(jnp.int32, _L)

    def chunk_body(c, carry):
        def vec_body(e, carry2):
            o = c * _CHUNK + e * _L
            b_loc = (ebase + o + iota) & (_BATCH - 1)
            users = plsc.load_gather(user_v, [b_loc])
            col = idx_v[pl.ds(o, _L)]
            flat_v[pl.ds(o, _L)] = col * _NUM_USERS + users
            return carry2

        lax.fori_loop(0, _VPC, vec_body, 0)
        pltpu.async_copy(
            pool_hbm.at[flat_v.at[pl.ds(c * _CHUNK, _CHUNK)]],
            out_v.at[pl.ds(c * _CHUNK, _CHUNK)],
            sem,
        )
        return carry

    lax.fori_loop(0, _NCH, chunk_body, 0)

    pltpu.make_async_copy(pool_hbm.at[pl.ds(0, _EPW)], out_v, sem).wait()

    pltpu.sync_copy(out_v, out_hbm.at[pl.ds(ebase, _EPW)])


def kernel(user_id, pool, idx_k):
    mesh = plsc.VectorSubcoreMesh(core_axis_name="c", subcore_axis_name="s")
    kfn = pl.kernel(
        _tec_body,
        mesh=mesh,
        compiler_params=pltpu.CompilerParams(
            use_tc_tiling_on_sc=False, needs_layout_passes=False),
        out_type=jax.ShapeDtypeStruct((_TOT,), jnp.int32),
        scratch_types=[
            pltpu.VMEM((_BATCH,), jnp.int32),
            pltpu.VMEM((_EPW,), jnp.int32),
            pltpu.VMEM((_EPW,), jnp.int32),
            pltpu.VMEM((_EPW,), jnp.int32),
            pltpu.SemaphoreType.DMA,
        ],
    )
    pool_lin_t = pool.T.reshape(-1)
    idxk_lin_t = idx_k.T.reshape(-1)
    neg_flat_t = kfn(user_id, pool_lin_t, idxk_lin_t)
    neg_items = neg_flat_t.reshape(_NUM_NEG, _BATCH).T
    log_q = jnp.full((_BATCH, _NUM_NEG), -math.log(float(_POOL_SIZE)),
                     dtype=jnp.float32)
    return neg_items, log_q
